# Initial kernel scaffold; baseline (speedup 1.0000x reference)
import jax
import jax.numpy as jnp
from jax.experimental import pallas as pl


def kernel(x, edge_index, pos_edge_index, neg_edge_index, W1, b1, W2, b2):
    raise NotImplementedError("write your pallas kernel here")


# speedup vs baseline: 6.7494x; 6.7494x over previous
import functools

import jax
import jax.numpy as jnp
from jax import lax
from jax.experimental import pallas as pl
from jax.experimental.pallas import tpu as pltpu
from jax.experimental.pallas import tpu_sc as plsc

N_PAD = 10240
D = 128
NC, NS = 2, 16
K1 = 79
K2 = 25
ROWS = N_PAD // NS
TC_NB = 1024

_MESH = plsc.VectorSubcoreMesh(core_axis_name="c", subcore_axis_name="s")


@functools.partial(
    pl.kernel, mesh=_MESH,
    compiler_params=pltpu.CompilerParams(needs_layout_passes=False),
    out_type=jax.ShapeDtypeStruct((NC, N_PAD, 16), jnp.float32),
    scratch_types=[
        pltpu.VMEM((K1, 128), jnp.int32),
        pltpu.VMEM((128, 16), jnp.float32),
        pltpu.VMEM_SHARED((N_PAD, 16), jnp.float32),
    ],
)
def _sc_deg(dst_hbm, zeros_hbm, ones_hbm, out_hbm, idx_v, ones_v, acc_sh):
    c = lax.axis_index("c")
    s = lax.axis_index("s")
    pltpu.sync_copy(dst_hbm.at[c, s], idx_v)
    pltpu.sync_copy(ones_hbm, ones_v)
    pltpu.sync_copy(zeros_hbm.at[pl.ds(s * ROWS, ROWS)],
                    acc_sh.at[pl.ds(s * ROWS, ROWS)])
    plsc.subcore_barrier()

    def body(j, carry):
        pltpu.sync_copy(ones_v, acc_sh.at[idx_v.at[j]], add=True)
        return carry

    lax.fori_loop(0, K1, body, 0)
    plsc.subcore_barrier()
    pltpu.sync_copy(acc_sh.at[pl.ds(s * ROWS, ROWS)],
                    out_hbm.at[c, pl.ds(s * ROWS, ROWS)])


@functools.partial(
    pl.kernel, mesh=_MESH,
    compiler_params=pltpu.CompilerParams(needs_layout_passes=False),
    out_type=jax.ShapeDtypeStruct((NC, N_PAD, D), jnp.float32),
    scratch_types=[
        pltpu.VMEM((K1, 128), jnp.int32),
        pltpu.VMEM((K1, 128), jnp.int32),
        pltpu.VMEM((128, D), jnp.float32),
        pltpu.VMEM_SHARED((N_PAD, D), jnp.float32),
        pltpu.SemaphoreType.DMA,
    ],
)
def _sc_gather_scatter(h_hbm, src_hbm, dst_hbm, zeros_hbm, out_hbm,
                       isrc, idst, gbuf, acc_sh, sem):
    c = lax.axis_index("c")
    s = lax.axis_index("s")
    pltpu.sync_copy(src_hbm.at[c, s], isrc)
    pltpu.sync_copy(dst_hbm.at[c, s], idst)
    pltpu.sync_copy(zeros_hbm.at[pl.ds(s * ROWS, ROWS)],
                    acc_sh.at[pl.ds(s * ROWS, ROWS)])
    plsc.subcore_barrier()

    def body(j, carry):
        pltpu.async_copy(h_hbm.at[isrc.at[j]], gbuf, sem).wait()
        pltpu.sync_copy(gbuf, acc_sh.at[idst.at[j]], add=True)
        return carry

    lax.fori_loop(0, K1, body, 0)
    plsc.subcore_barrier()
    pltpu.sync_copy(acc_sh.at[pl.ds(s * ROWS, ROWS)],
                    out_hbm.at[c, pl.ds(s * ROWS, ROWS)])


@functools.partial(
    pl.kernel, mesh=_MESH,
    compiler_params=pltpu.CompilerParams(needs_layout_passes=False),
    out_type=jax.ShapeDtypeStruct((NC, NS, K2, 128), jnp.float32),
    scratch_types=[
        pltpu.VMEM((K2, 128), jnp.int32),
        pltpu.VMEM((K2, 128), jnp.int32),
        pltpu.VMEM((128, D), jnp.float32),
        pltpu.VMEM((128, D), jnp.float32),
        pltpu.VMEM((K2, 128), jnp.float32),
        pltpu.SemaphoreType.DMA,
    ],
)
def _sc_decode(z_hbm, src_hbm, dst_hbm, out_hbm, isrc, idst, abuf, bbuf,
               sbuf, sem):
    c = lax.axis_index("c")
    s = lax.axis_index("s")
    pltpu.sync_copy(src_hbm.at[c, s], isrc)
    pltpu.sync_copy(dst_hbm.at[c, s], idst)
    lane = lax.iota(jnp.int32, 16)

    def chunk(j, carry):
        pltpu.async_copy(z_hbm.at[isrc.at[j]], abuf, sem).wait()
        pltpu.async_copy(z_hbm.at[idst.at[j]], bbuf, sem).wait()
        for g in range(8):
            rows = lane + g * 16

            def col_blk(cb, acc):
                for t in range(16):
                    cols = lane * 0 + (cb * 16 + t)
                    av = plsc.load_gather(abuf, [rows, cols])
                    bv = plsc.load_gather(bbuf, [rows, cols])
                    acc = acc + av * bv
                return acc

            acc = lax.fori_loop(0, 8, col_blk, jnp.zeros((16,), jnp.float32))
            sbuf[j, pl.ds(g * 16, 16)] = 1.0 / (1.0 + jnp.exp(-acc))
        return carry

    lax.fori_loop(0, K2, chunk, 0)
    pltpu.sync_copy(sbuf, out_hbm.at[c, s])


def _dinv(deg_ref):
    deg = deg_ref[0, :, :1] + deg_ref[1, :, :1] + 1.0
    return lax.rsqrt(deg)


def _tc_a_body(x_ref, w_ref, deg_ref, o_ref):
    h = jnp.dot(x_ref[...], w_ref[...], preferred_element_type=jnp.float32)
    o_ref[...] = h * _dinv(deg_ref)


def _tc_b_body(acc_ref, h_ref, b_ref, w_ref, deg_ref, o_ref):
    dinv = _dinv(deg_ref)
    y = jnp.maximum(dinv * (acc_ref[0] + acc_ref[1] + h_ref[...]) + b_ref[...],
                    0.0)
    o_ref[...] = dinv * jnp.dot(y, w_ref[...],
                                preferred_element_type=jnp.float32)


def _tc_c_body(acc_ref, h_ref, b_ref, deg_ref, o_ref):
    dinv = _dinv(deg_ref)
    o_ref[...] = dinv * (acc_ref[0] + acc_ref[1] + h_ref[...]) + b_ref[...]


_ROW_SPEC = pl.BlockSpec((TC_NB, D), lambda i: (i, 0))
_ACC_SPEC = pl.BlockSpec((2, TC_NB, D), lambda i: (0, i, 0))
_DEG_SPEC = pl.BlockSpec((2, TC_NB, 16), lambda i: (0, i, 0))
_W_SPEC = pl.BlockSpec((D, D), lambda i: (0, 0))
_B_SPEC = pl.BlockSpec((1, D), lambda i: (0, 0))
_GRID = (N_PAD // TC_NB,)
_OUT_T = jax.ShapeDtypeStruct((N_PAD, D), jnp.float32)


def _tc_a(xp, W1, deg2):
    return pl.pallas_call(
        _tc_a_body, grid=_GRID,
        in_specs=[_ROW_SPEC, _W_SPEC, _DEG_SPEC],
        out_specs=_ROW_SPEC, out_shape=_OUT_T)(xp, W1, deg2)


def _tc_b(acc1, h1p, b1r, W2, deg2):
    return pl.pallas_call(
        _tc_b_body, grid=_GRID,
        in_specs=[_ACC_SPEC, _ROW_SPEC, _B_SPEC, _W_SPEC, _DEG_SPEC],
        out_specs=_ROW_SPEC, out_shape=_OUT_T)(acc1, h1p, b1r, W2, deg2)


def _tc_c(acc2, h2p, b2r, deg2):
    return pl.pallas_call(
        _tc_c_body, grid=_GRID,
        in_specs=[_ACC_SPEC, _ROW_SPEC, _B_SPEC, _DEG_SPEC],
        out_specs=_ROW_SPEC, out_shape=_OUT_T)(acc2, h2p, b2r, deg2)


def kernel(x, edge_index, pos_edge_index, neg_edge_index, W1, b1, W2, b2):
    n = x.shape[0]
    e = edge_index.shape[1]
    pe = pos_edge_index.shape[1]

    ep = NC * NS * K1 * 128
    srcp = jnp.concatenate(
        [edge_index[0], jnp.zeros((ep - e,), jnp.int32)]).reshape(NC, NS, K1, 128)
    dstp = jnp.concatenate(
        [edge_index[1], jnp.full((ep - e,), n, jnp.int32)]).reshape(NC, NS, K1, 128)

    xp = jnp.pad(x, ((0, N_PAD - n), (0, 0)))
    zeros_nd = jnp.zeros((N_PAD, D), jnp.float32)
    zeros_16 = jnp.zeros((N_PAD, 16), jnp.float32)
    ones_128 = jnp.ones((128, 16), jnp.float32)
    b1r = b1.reshape(1, D)
    b2r = b2.reshape(1, D)

    deg2 = _sc_deg(dstp, zeros_16, ones_128)
    h1p = _tc_a(xp, W1, deg2)
    acc1 = _sc_gather_scatter(h1p, srcp, dstp, zeros_nd)
    h2p = _tc_b(acc1, h1p, b1r, W2, deg2)
    acc2 = _sc_gather_scatter(h2p, srcp, dstp, zeros_nd)
    z = _tc_c(acc2, h2p, b2r, deg2)

    ce = 2 * pe
    ep2 = NC * NS * K2 * 128
    dsrc = jnp.concatenate(
        [pos_edge_index[0], neg_edge_index[0],
         jnp.zeros((ep2 - ce,), jnp.int32)]).reshape(NC, NS, K2, 128)
    ddst = jnp.concatenate(
        [pos_edge_index[1], neg_edge_index[1],
         jnp.zeros((ep2 - ce,), jnp.int32)]).reshape(NC, NS, K2, 128)
    scores = _sc_decode(z, dsrc, ddst).reshape(-1)
    return scores[:pe], scores[pe:ce]
